# baseline (speedup 1.0000x reference)
import jax
import jax.numpy as jnp
from jax.experimental import pallas as pl

_M = 512
_K = 8193
_N = 2049
_KM = 8192
_NT = 128


def _mm_body(x_ref, w_ref, xl_ref, wl_ref, o_ref):
    wb = w_ref[...].astype(jnp.bfloat16)
    acc = jax.lax.dot_general(
        x_ref[...], wb, (((1,), (0,)), ((), ())),
        preferred_element_type=jnp.float32)
    o_ref[...] = acc + xl_ref[...] * wl_ref[...]


def kernel(x_affine, W):
    x_bf = x_affine[:, :_KM].astype(jnp.bfloat16)
    x_last = x_affine[:, _KM:]
    w_last = W[_KM:, :]
    grid = (pl.cdiv(_N, _NT),)
    return pl.pallas_call(
        _mm_body,
        grid=grid,
        in_specs=[
            pl.BlockSpec((_M, _KM), lambda j: (0, 0)),
            pl.BlockSpec((_KM, _NT), lambda j: (0, j)),
            pl.BlockSpec((_M, 1), lambda j: (0, 0)),
            pl.BlockSpec((1, _NT), lambda j: (0, j)),
        ],
        out_specs=pl.BlockSpec((_M, _NT), lambda j: (0, j)),
        out_shape=jax.ShapeDtypeStruct((_M, _N), jnp.float32),
    )(x_bf, W, x_last, w_last)


# speedup vs baseline: 1.0963x; 1.0963x over previous
import jax
import jax.numpy as jnp
from jax.experimental import pallas as pl

_M = 512
_K = 8193
_N = 2049
_KM = 8192
_NT = 512


def _mm_body(x_ref, w_ref, xl_ref, wl_ref, o_ref):
    wb = w_ref[...].astype(jnp.bfloat16)
    acc = jax.lax.dot_general(
        x_ref[...], wb, (((1,), (0,)), ((), ())),
        preferred_element_type=jnp.float32)
    o_ref[...] = acc + xl_ref[...] * wl_ref[...]


def kernel(x_affine, W):
    x_bf = x_affine[:, :_KM].astype(jnp.bfloat16)
    x_last = x_affine[:, _KM:]
    w_last = W[_KM:, :]
    grid = (pl.cdiv(_N, _NT),)
    return pl.pallas_call(
        _mm_body,
        grid=grid,
        in_specs=[
            pl.BlockSpec((_M, _KM), lambda j: (0, 0)),
            pl.BlockSpec((_KM, _NT), lambda j: (0, j)),
            pl.BlockSpec((_M, 1), lambda j: (0, 0)),
            pl.BlockSpec((1, _NT), lambda j: (0, j)),
        ],
        out_specs=pl.BlockSpec((_M, _NT), lambda j: (0, j)),
        out_shape=jax.ShapeDtypeStruct((_M, _N), jnp.float32),
    )(x_bf, W, x_last, w_last)
